# baseline (speedup 1.0000x reference)
import functools

import jax
import jax.numpy as jnp
from jax import lax
from jax.experimental import pallas as pl
from jax.experimental.pallas import tpu as pltpu
from jax.experimental.pallas import tpu_sc as plsc

NC = 2
NS = 16
L = 16
C = 32
FW = 128


def _prep_body(x_ref, w_ref, att_ref, hext_ref, as_ref, ad_ref):
    xb = x_ref[...]
    W = w_ref[0]
    att = att_ref[0]
    h = jnp.dot(xb, W, preferred_element_type=jnp.float32)
    a_s = jnp.dot(h, att[:128, :], preferred_element_type=jnp.float32)
    a_d = jnp.dot(h, att[128:, :], preferred_element_type=jnp.float32)
    hext_ref[0] = h
    as_ref[0] = a_s
    ad_ref[0] = a_d


def _combine_body(acc_ref, out_ref):
    out_ref[...] = jnp.maximum(acc_ref[0] + acc_ref[1], 0.0)


def _make_sc_edges(n, nchunk, ep):
    mesh = plsc.VectorSubcoreMesh(
        core_axis_name="c", subcore_axis_name="s", num_cores=NC, num_subcores=NS
    )
    rows_per = ((-(-n // NS) + 127) // 128) * 128
    n_acc = rows_per * NS
    nfull = rows_per // C
    assert nchunk % 4 == 0 and rows_per % C == 0

    @functools.partial(
        pl.kernel,
        out_type=pltpu.HBM((NC, n_acc, FW), jnp.float32),
        mesh=mesh,
        compiler_params=pltpu.CompilerParams(needs_layout_passes=False),
        scratch_types=[
            pltpu.VMEM((n,), jnp.float32),
            pltpu.VMEM((n,), jnp.float32),
            [pltpu.VMEM((C,), jnp.int32) for _ in range(4)],
            [pltpu.VMEM((C,), jnp.int32) for _ in range(4)],
            [pltpu.VMEM((C, FW), jnp.float32) for _ in range(2)],
            [pltpu.VMEM((C, FW), jnp.float32) for _ in range(2)],
            [pltpu.VMEM((C,), jnp.float32) for _ in range(2)],
            pltpu.VMEM((rows_per,), jnp.float32),
            pltpu.VMEM_SHARED((n_acc, FW), jnp.float32),
            pltpu.VMEM_SHARED((n_acc,), jnp.float32),
            [pltpu.SemaphoreType.DMA for _ in range(2)],
            [pltpu.SemaphoreType.DMA for _ in range(2)],
            [pltpu.SemaphoreType.DMA for _ in range(2)],
            pltpu.SemaphoreType.DMA,
            pltpu.SemaphoreType.DMA,
        ],
    )
    def sc_edges(hext_hbm, as_hbm, ad_hbm, s_hbm, t_hbm, out_hbm,
                 as_v, ad_v, six, tix, gb, sbv, wbuf, dsum,
                 num_sh, den_sh, gsem, ssem, dsem, fsem_s, fsem_t):
        tid = lax.axis_index("s")
        base = tid * rows_per
        lanes = lax.iota(jnp.int32, L)

        def branch_body(br, carry):
            pltpu.sync_copy(as_hbm.at[br], as_v)
            pltpu.sync_copy(ad_hbm.at[br], ad_v)

            def zrow(r, c2):
                for j in range(FW // L):
                    gb[0][r, pl.ds(L * j, L)] = jnp.zeros((L,), jnp.float32)
                return c2
            lax.fori_loop(0, C, zrow, None)
            for k in range(nfull):
                pltpu.sync_copy(gb[0], num_sh.at[pl.ds(base + k * C, C)])

            def dzero(k, c2):
                dsum[pl.ds(k * L, L)] = jnp.zeros((L,), jnp.float32)
                return c2
            lax.fori_loop(0, rows_per // L, dzero, None)
            pltpu.sync_copy(dsum, den_sh.at[pl.ds(base, rows_per)])

            plsc.subcore_barrier()

            off = br * n

            def fetch_idx(c, slot):
                pltpu.async_copy(
                    s_hbm.at[br, tid, 0, pl.ds(c * C, C)], six[slot], fsem_s)
                pltpu.async_copy(
                    t_hbm.at[br, tid, 0, pl.ds(c * C, C)], tix[slot], fsem_t)

            def wait_idx(c, slot):
                pltpu.make_async_copy(
                    s_hbm.at[br, tid, 0, pl.ds(c * C, C)], six[slot],
                    fsem_s).wait()
                pltpu.make_async_copy(
                    t_hbm.at[br, tid, 0, pl.ds(c * C, C)], tix[slot],
                    fsem_t).wait()

            for b in range(2):
                fetch_idx(b, b)
                wait_idx(b, b)
                pltpu.async_copy(hext_hbm.at[tix[b]], gb[b], gsem[b])

            def quad(i, carry2):
                for b in range(4):
                    c = 4 * i + b
                    rb = b % 2
                    pltpu.make_async_copy(
                        hext_hbm.at[tix[b]], gb[rb], gsem[rb]).wait()

                    @pl.when(c >= 2)
                    def _wait_prev_scatter():
                        pltpu.make_async_copy(
                            sbv[rb], num_sh.at[six[(b + 2) % 4]],
                            ssem[rb]).wait()
                        pltpu.make_async_copy(
                            wbuf[rb], den_sh.at[six[(b + 2) % 4]],
                            dsem[rb]).wait()

                    @pl.when(c + 2 < nchunk)
                    def _fetch_next_idx():
                        fetch_idx(c + 2, (b + 2) % 4)

                    @plsc.parallel_loop(0, C // L, 1, unroll=C // L)
                    def srow(qi):
                        q = qi * L
                        sv = six[b][pl.ds(q, L)]
                        tv = tix[b][pl.ds(q, L)] - off
                        a1 = plsc.load_gather(as_v, [sv])
                        a2 = plsc.load_gather(ad_v, [tv])
                        e = jnp.maximum(a1 + a2, 0.0)
                        w = jnp.where(c * C + q + lanes < ep, jnp.exp(e), 0.0)
                        wbuf[rb][pl.ds(q, L)] = w
                        for lane in range(L):
                            ws = w[lane]
                            r = q + lane
                            for j in range(FW // L):
                                sl = pl.ds(L * j, L)
                                sbv[rb][r, sl] = gb[rb][r, sl] * ws

                    pltpu.async_copy(
                        sbv[rb], num_sh.at[six[b]], ssem[rb], add=True)
                    pltpu.async_copy(
                        wbuf[rb], den_sh.at[six[b]], dsem[rb], add=True)

                    @pl.when(c + 2 < nchunk)
                    def _next_gather():
                        wait_idx(c + 2, (b + 2) % 4)
                        pltpu.async_copy(
                            hext_hbm.at[tix[(b + 2) % 4]], gb[rb], gsem[rb])
                return carry2
            lax.fori_loop(0, nchunk // 4, quad, None)

            for rb in range(2):
                pltpu.make_async_copy(
                    sbv[rb], num_sh.at[six[rb]], ssem[rb]).wait()
                pltpu.make_async_copy(
                    wbuf[rb], den_sh.at[six[rb]], dsem[rb]).wait()

            plsc.subcore_barrier()

            pltpu.sync_copy(den_sh.at[pl.ds(base, rows_per)], dsum)

            def dinv(j, c2):
                sl = pl.ds(j * L, L)
                dsum[sl] = 1.0 / (dsum[sl] + 1e-10)
                return c2
            lax.fori_loop(0, rows_per // L, dinv, None)

            for k in range(nfull):
                pltpu.sync_copy(num_sh.at[pl.ds(base + k * C, C)], gb[0])

                @plsc.parallel_loop(0, C // L, 1, unroll=C // L)
                def nrow(qi):
                    q = qi * L
                    wvec = dsum[pl.ds(k * C + q, L)]
                    for lane in range(L):
                        ws = wvec[lane]
                        r = q + lane
                        for j in range(FW // L):
                            sl = pl.ds(L * j, L)
                            sbv[0][r, sl] = gb[0][r, sl] * ws
                pltpu.sync_copy(sbv[0], out_hbm.at[br, pl.ds(base + k * C, C)])

            plsc.subcore_barrier()
            return carry

        branch_body(lax.axis_index("c"), None)

    return sc_edges


def kernel(x, edge_index_do, edge_index_up, Wirr, Wsol, Wskip, att_irr, att_sol):
    del Wskip
    n, fin = x.shape
    e = edge_index_do.shape[1]
    blk = 1000
    gn = n // blk

    wstack = jnp.stack([Wirr, Wsol])
    attstack = jnp.stack([att_irr, att_sol])

    hext, a_src, a_dst = pl.pallas_call(
        _prep_body,
        grid=(NC, gn),
        in_specs=[
            pl.BlockSpec((blk, fin), lambda b, i: (i, 0)),
            pl.BlockSpec((1, fin, 128), lambda b, i: (b, 0, 0)),
            pl.BlockSpec((1, 256, 1), lambda b, i: (b, 0, 0)),
        ],
        out_specs=[
            pl.BlockSpec((1, blk, FW), lambda b, i: (b, i, 0)),
            pl.BlockSpec((1, blk, 1), lambda b, i: (b, i, 0)),
            pl.BlockSpec((1, blk, 1), lambda b, i: (b, i, 0)),
        ],
        out_shape=[
            jax.ShapeDtypeStruct((NC, n, FW), jnp.float32),
            jax.ShapeDtypeStruct((NC, n, 1), jnp.float32),
            jax.ShapeDtypeStruct((NC, n, 1), jnp.float32),
        ],
    )(x, wstack, attstack)

    ep = e // NS
    nchunk = -(-ep // C)
    nchunk += (-nchunk) % 4
    pad = nchunk * C - ep
    s_all = jnp.stack([edge_index_do[0], edge_index_up[0]]).reshape(NC, NS, ep)
    t_all = jnp.stack([edge_index_do[1], edge_index_up[1] + n]).reshape(NC, NS, ep)
    s_pad = jnp.pad(s_all, ((0, 0), (0, 0), (0, pad)),
                    mode="edge").reshape(NC, NS, 1, nchunk * C)
    t_pad = jnp.pad(t_all, ((0, 0), (0, 0), (0, pad)),
                    mode="edge").reshape(NC, NS, 1, nchunk * C)

    acc = _make_sc_edges(n, nchunk, ep)(
        hext.reshape(NC * n, FW), a_src.reshape(NC, n), a_dst.reshape(NC, n),
        s_pad, t_pad)

    out = pl.pallas_call(
        _combine_body,
        grid=(gn,),
        in_specs=[pl.BlockSpec((NC, blk, FW), lambda i: (0, i, 0))],
        out_specs=pl.BlockSpec((blk, 128), lambda i: (i, 0)),
        out_shape=jax.ShapeDtypeStruct((n, 128), jnp.float32),
    )(acc)
    return out


# speedup vs baseline: 1.1209x; 1.1209x over previous
import functools

import jax
import jax.numpy as jnp
from jax import lax
from jax.experimental import pallas as pl
from jax.experimental.pallas import tpu as pltpu
from jax.experimental.pallas import tpu_sc as plsc

NC = 2
NS = 16
L = 16
C = 32
FW = 128
DD = 3
ID = 6


def _prep_body(x_ref, w_ref, att_ref, hext_ref, as_ref, ad_ref):
    xb = x_ref[...]
    W = w_ref[0]
    att = att_ref[0]
    h = jnp.dot(xb, W, preferred_element_type=jnp.float32)
    a_s = jnp.dot(h, att[:128, :], preferred_element_type=jnp.float32)
    a_d = jnp.dot(h, att[128:, :], preferred_element_type=jnp.float32)
    hext_ref[0] = h
    as_ref[0] = a_s
    ad_ref[0] = a_d


def _combine_body(acc_ref, out_ref):
    out_ref[...] = jnp.maximum(acc_ref[0] + acc_ref[1], 0.0)


def _make_sc_edges(n, nchunk, ep):
    mesh = plsc.VectorSubcoreMesh(
        core_axis_name="c", subcore_axis_name="s", num_cores=NC, num_subcores=NS
    )
    rows_per = ((-(-n // NS) + 127) // 128) * 128
    n_acc = rows_per * NS
    nfull = rows_per // C
    assert nchunk % ID == 0 and rows_per % C == 0

    @functools.partial(
        pl.kernel,
        out_type=pltpu.HBM((NC, n_acc, FW), jnp.float32),
        mesh=mesh,
        compiler_params=pltpu.CompilerParams(needs_layout_passes=False),
        scratch_types=[
            pltpu.VMEM((n,), jnp.float32),
            pltpu.VMEM((n,), jnp.float32),
            [pltpu.VMEM((C,), jnp.int32) for _ in range(ID)],
            [pltpu.VMEM((C,), jnp.int32) for _ in range(ID)],
            [pltpu.VMEM((C,), jnp.int32) for _ in range(DD)],
            [pltpu.VMEM((C, FW), jnp.float32) for _ in range(DD)],
            [pltpu.VMEM((C, FW), jnp.float32) for _ in range(DD)],
            [pltpu.VMEM((C,), jnp.float32) for _ in range(DD)],
            pltpu.VMEM((rows_per,), jnp.float32),
            pltpu.VMEM_SHARED((n_acc, FW), jnp.float32),
            pltpu.VMEM_SHARED((n_acc,), jnp.float32),
            [pltpu.SemaphoreType.DMA for _ in range(DD)],
            [pltpu.SemaphoreType.DMA for _ in range(DD)],
            [pltpu.SemaphoreType.DMA for _ in range(DD)],
            [pltpu.SemaphoreType.DMA for _ in range(ID)],
        ],
    )
    def sc_edges(hext_hbm, as_hbm, ad_hbm, s_hbm, t_hbm, out_hbm,
                 as_v, ad_v, six, tix, ssix, gb, sbv, wbuf, dsum,
                 num_sh, den_sh, gsem, ssem, dsem, isem):
        tid = lax.axis_index("s")
        base = tid * rows_per
        lanes = lax.iota(jnp.int32, L)

        def branch_body(br, carry):
            pltpu.sync_copy(as_hbm.at[br], as_v)
            pltpu.sync_copy(ad_hbm.at[br], ad_v)

            def zrow(r, c2):
                for j in range(FW // L):
                    gb[0][r, pl.ds(L * j, L)] = jnp.zeros((L,), jnp.float32)
                return c2
            lax.fori_loop(0, C, zrow, None)
            for k in range(nfull):
                pltpu.sync_copy(gb[0], num_sh.at[pl.ds(base + k * C, C)])

            def dzero(k, c2):
                dsum[pl.ds(k * L, L)] = jnp.zeros((L,), jnp.float32)
                return c2
            lax.fori_loop(0, rows_per // L, dzero, None)
            pltpu.sync_copy(dsum, den_sh.at[pl.ds(base, rows_per)])

            plsc.subcore_barrier()

            off = br * n

            def fetch_idx(c, slot):
                pltpu.async_copy(
                    s_hbm.at[br, tid, 0, pl.ds(c * C, C)], six[slot],
                    isem[slot])
                pltpu.async_copy(
                    t_hbm.at[br, tid, 0, pl.ds(c * C, C)], tix[slot],
                    isem[slot])

            def wait_idx(c, slot):
                pltpu.make_async_copy(
                    s_hbm.at[br, tid, 0, pl.ds(c * C, C)], six[slot],
                    isem[slot]).wait()
                pltpu.make_async_copy(
                    t_hbm.at[br, tid, 0, pl.ds(c * C, C)], tix[slot],
                    isem[slot]).wait()

            for k in range(ID - 1):
                fetch_idx(k, k)
            for k in range(DD):
                wait_idx(k, k)
                pltpu.async_copy(hext_hbm.at[tix[k]], gb[k], gsem[k])

            def group(i, carry2):
                for b in range(ID):
                    c = ID * i + b
                    rb = b % DD
                    pltpu.make_async_copy(
                        hext_hbm.at[tix[b]], gb[rb], gsem[rb]).wait()

                    @pl.when(c >= DD)
                    def _wait_prev_scatter():
                        pltpu.make_async_copy(
                            sbv[rb], num_sh.at[ssix[rb]], ssem[rb]).wait()
                        pltpu.make_async_copy(
                            wbuf[rb], den_sh.at[ssix[rb]], dsem[rb]).wait()

                    @pl.when(c + ID - 1 < nchunk)
                    def _fetch_next_idx():
                        fetch_idx(c + ID - 1, (b + ID - 1) % ID)

                    @plsc.parallel_loop(0, C // L, 1, unroll=C // L)
                    def srow(qi):
                        q = qi * L
                        sv = six[b][pl.ds(q, L)]
                        tv = tix[b][pl.ds(q, L)] - off
                        ssix[rb][pl.ds(q, L)] = sv
                        a1 = plsc.load_gather(as_v, [sv])
                        a2 = plsc.load_gather(ad_v, [tv])
                        e = jnp.maximum(a1 + a2, 0.0)
                        w = jnp.where(c * C + q + lanes < ep, jnp.exp(e), 0.0)
                        wbuf[rb][pl.ds(q, L)] = w
                        for lane in range(L):
                            ws = w[lane]
                            r = q + lane
                            for j in range(FW // L):
                                sl = pl.ds(L * j, L)
                                sbv[rb][r, sl] = gb[rb][r, sl] * ws

                    pltpu.async_copy(
                        sbv[rb], num_sh.at[ssix[rb]], ssem[rb], add=True)
                    pltpu.async_copy(
                        wbuf[rb], den_sh.at[ssix[rb]], dsem[rb], add=True)

                    @pl.when(c + DD < nchunk)
                    def _next_gather():
                        wait_idx(c + DD, (b + DD) % ID)
                        pltpu.async_copy(
                            hext_hbm.at[tix[(b + DD) % ID]], gb[rb], gsem[rb])
                return carry2
            lax.fori_loop(0, nchunk // ID, group, None)

            for rb in range(DD):
                pltpu.make_async_copy(
                    sbv[rb], num_sh.at[ssix[rb]], ssem[rb]).wait()
                pltpu.make_async_copy(
                    wbuf[rb], den_sh.at[ssix[rb]], dsem[rb]).wait()

            plsc.subcore_barrier()

            pltpu.sync_copy(den_sh.at[pl.ds(base, rows_per)], dsum)

            def dinv(j, c2):
                sl = pl.ds(j * L, L)
                dsum[sl] = 1.0 / (dsum[sl] + 1e-10)
                return c2
            lax.fori_loop(0, rows_per // L, dinv, None)

            for k in range(nfull):
                pltpu.sync_copy(num_sh.at[pl.ds(base + k * C, C)], gb[0])

                @plsc.parallel_loop(0, C // L, 1, unroll=C // L)
                def nrow(qi):
                    q = qi * L
                    wvec = dsum[pl.ds(k * C + q, L)]
                    for lane in range(L):
                        ws = wvec[lane]
                        r = q + lane
                        for j in range(FW // L):
                            sl = pl.ds(L * j, L)
                            sbv[0][r, sl] = gb[0][r, sl] * ws
                pltpu.sync_copy(sbv[0], out_hbm.at[br, pl.ds(base + k * C, C)])

            plsc.subcore_barrier()
            return carry

        branch_body(lax.axis_index("c"), None)

    return sc_edges


def kernel(x, edge_index_do, edge_index_up, Wirr, Wsol, Wskip, att_irr, att_sol):
    del Wskip
    n, fin = x.shape
    e = edge_index_do.shape[1]
    blk = 1000
    gn = n // blk

    wstack = jnp.stack([Wirr, Wsol])
    attstack = jnp.stack([att_irr, att_sol])

    hext, a_src, a_dst = pl.pallas_call(
        _prep_body,
        grid=(NC, gn),
        in_specs=[
            pl.BlockSpec((blk, fin), lambda b, i: (i, 0)),
            pl.BlockSpec((1, fin, 128), lambda b, i: (b, 0, 0)),
            pl.BlockSpec((1, 256, 1), lambda b, i: (b, 0, 0)),
        ],
        out_specs=[
            pl.BlockSpec((1, blk, FW), lambda b, i: (b, i, 0)),
            pl.BlockSpec((1, blk, 1), lambda b, i: (b, i, 0)),
            pl.BlockSpec((1, blk, 1), lambda b, i: (b, i, 0)),
        ],
        out_shape=[
            jax.ShapeDtypeStruct((NC, n, FW), jnp.float32),
            jax.ShapeDtypeStruct((NC, n, 1), jnp.float32),
            jax.ShapeDtypeStruct((NC, n, 1), jnp.float32),
        ],
    )(x, wstack, attstack)

    ep = e // NS
    nchunk = -(-ep // C)
    nchunk += (-nchunk) % ID
    pad = nchunk * C - ep
    s_all = jnp.stack([edge_index_do[0], edge_index_up[0]]).reshape(NC, NS, ep)
    t_all = jnp.stack([edge_index_do[1], edge_index_up[1] + n]).reshape(NC, NS, ep)
    s_pad = jnp.pad(s_all, ((0, 0), (0, 0), (0, pad)),
                    mode="edge").reshape(NC, NS, 1, nchunk * C)
    t_pad = jnp.pad(t_all, ((0, 0), (0, 0), (0, pad)),
                    mode="edge").reshape(NC, NS, 1, nchunk * C)

    acc = _make_sc_edges(n, nchunk, ep)(
        hext.reshape(NC * n, FW), a_src.reshape(NC, n), a_dst.reshape(NC, n),
        s_pad, t_pad)

    out = pl.pallas_call(
        _combine_body,
        grid=(gn,),
        in_specs=[pl.BlockSpec((NC, blk, FW), lambda i: (0, i, 0))],
        out_specs=pl.BlockSpec((blk, 128), lambda i: (i, 0)),
        out_shape=jax.ShapeDtypeStruct((n, 128), jnp.float32),
    )(acc)
    return out
